# Initial kernel scaffold; baseline (speedup 1.0000x reference)
import jax
import jax.numpy as jnp
from jax.experimental import pallas as pl


def kernel(x, encoder, encoder_bias, decoder, decoder_bias):
    raise NotImplementedError("write your pallas kernel here")


# speedup vs baseline: 3.7882x; 3.7882x over previous
import functools

import jax
import jax.numpy as jnp
from jax.experimental import pallas as pl
from jax.experimental.pallas import tpu as pltpu

TOPK = 32


def _encode_body(x_ref, enc_ref, bias_ref, out_ref):
    out_ref[...] = (
        jnp.dot(x_ref[...], enc_ref[...], preferred_element_type=jnp.float32)
        + bias_ref[...]
    )


def _threshold_body(h_ref, thr_ref):
    h = h_ref[...]
    bi = jax.lax.bitcast_convert_type(h, jnp.int32)
    bu = jax.lax.bitcast_convert_type(h, jnp.uint32)
    u = jnp.where(bi < 0, ~bu, bu | jnp.uint32(0x80000000))

    def step(i, cand):
        bit = jnp.uint32(1) << (jnp.uint32(31) - i)
        t = cand | bit
        cnt = jnp.sum((u >= t).astype(jnp.int32), axis=1, keepdims=True)
        return jnp.where(cnt >= TOPK, t, cand)

    cand = jax.lax.fori_loop(
        0, 32, step, jnp.zeros((h.shape[0], 1), jnp.uint32), unroll=True
    )
    back = jnp.where(
        cand >= jnp.uint32(0x80000000), cand & jnp.uint32(0x7FFFFFFF), ~cand
    )
    thr = jax.lax.bitcast_convert_type(back, jnp.float32)
    thr_ref[...] = jnp.broadcast_to(thr, thr_ref.shape)


def _decode_body(h_ref, thr_ref, dec_ref, dbias_ref, out_ref):
    k = pl.program_id(1)
    thr = thr_ref[:, 0:1]
    a = h_ref[...]
    a = jnp.where(a >= thr, a, 0.0)
    a = jnp.maximum(a, 0.0)

    @pl.when(k == 0)
    def _():
        out_ref[...] = jnp.broadcast_to(dbias_ref[...], out_ref.shape)

    out_ref[...] += jnp.dot(a, dec_ref[...], preferred_element_type=jnp.float32)


def kernel(x, encoder, encoder_bias, decoder, decoder_bias):
    B, D = x.shape
    H = encoder.shape[1]

    M_BLK = min(1024, B)
    H_BLK = min(512, H)
    TB = min(64, B)

    h = pl.pallas_call(
        _encode_body,
        grid=(B // M_BLK, H // H_BLK),
        in_specs=[
            pl.BlockSpec((M_BLK, D), lambda m, hb: (m, 0)),
            pl.BlockSpec((D, H_BLK), lambda m, hb: (0, hb)),
            pl.BlockSpec((1, H_BLK), lambda m, hb: (0, hb)),
        ],
        out_specs=pl.BlockSpec((M_BLK, H_BLK), lambda m, hb: (m, hb)),
        out_shape=jax.ShapeDtypeStruct((B, H), jnp.float32),
    )(x, encoder, encoder_bias.reshape(1, H))

    thr = pl.pallas_call(
        _threshold_body,
        grid=(B // TB,),
        in_specs=[pl.BlockSpec((TB, H), lambda tb: (tb, 0))],
        out_specs=pl.BlockSpec((TB, 128), lambda tb: (tb, 0)),
        out_shape=jax.ShapeDtypeStruct((B, 128), jnp.float32),
    )(h)

    recon = pl.pallas_call(
        _decode_body,
        grid=(B // M_BLK, H // H_BLK),
        in_specs=[
            pl.BlockSpec((M_BLK, H_BLK), lambda m, k: (m, k)),
            pl.BlockSpec((M_BLK, 128), lambda m, k: (m, 0)),
            pl.BlockSpec((H_BLK, D), lambda m, k: (k, 0)),
            pl.BlockSpec((1, D), lambda m, k: (0, 0)),
        ],
        out_specs=pl.BlockSpec((M_BLK, D), lambda m, k: (m, 0)),
        out_shape=jax.ShapeDtypeStruct((B, D), jnp.float32),
        compiler_params=pltpu.CompilerParams(
            dimension_semantics=("parallel", "arbitrary")
        ),
    )(h, thr, decoder, decoder_bias.reshape(1, D))

    return recon
